# baseline (speedup 1.0000x reference)
import functools

import jax
import jax.numpy as jnp
from jax import lax
from jax.experimental import pallas as pl
from jax.experimental.pallas import tpu as pltpu
from jax.experimental.pallas import tpu_sc as plsc

DIM = 64
DIMP = 128
B_ROWS = 16384
B_COLS = 50
VOCAB = 1000000
TILE = 128
NFULL = VOCAB // TILE
TAIL = VOCAB - NFULL * TILE
VOCABP = (NFULL + 1) * TILE

_info = plsc.get_sparse_core_info()
NC = _info.num_cores
NS = _info.num_subcores
NW = NC * NS

_MESH = dict(mesh=plsc.VectorSubcoreMesh(core_axis_name="c", subcore_axis_name="s"))
_PARAMS = pltpu.CompilerParams(use_tc_tiling_on_sc=True,
                               needs_layout_passes=False)

ROUNDS = NFULL // NW
EXTRA = NFULL - ROUNDS * NW


@functools.partial(
    pl.kernel,
    **_MESH,
    out_type=jax.ShapeDtypeStruct((VOCABP, DIMP), jnp.float32),
    scratch_types=[
        pltpu.VMEM((2, DIM, TILE), jnp.float32),
        pltpu.VMEM((2, TILE, DIMP + 1), jnp.float32),
        pltpu.SemaphoreType.DMA,
        pltpu.SemaphoreType.DMA,
    ],
    compiler_params=_PARAMS,
)
def _relayout_kernel(wt_hbm, wtail_hbm, tbl_hbm, blk_v, out_v, in_sem, out_sem):
    wid = lax.axis_index("s") * NC + lax.axis_index("c")
    iotas = [lax.iota(jnp.int32, 16) + 16 * j for j in range(8)]

    def stage(t, buf):
        pltpu.async_copy(wt_hbm.at[:, pl.ds(t * TILE, TILE)], blk_v.at[buf],
                         in_sem)

    def wait_stage(buf):
        pltpu.make_async_copy(wt_hbm.at[:, pl.ds(0, TILE)], blk_v.at[buf],
                              in_sem).wait()

    def transpose(buf):
        def comp(c, _):
            csplat = jnp.full((16,), c, jnp.int32)
            for j in range(8):
                v = blk_v[buf, c, pl.ds(16 * j, 16)]
                plsc.store_scatter(out_v.at[buf], [iotas[j], csplat], v)
            return _

        lax.fori_loop(0, DIM, comp, None, unroll=4)

    def writeback(t, buf):
        pltpu.async_copy(out_v.at[buf, :, pl.ds(0, DIMP)],
                         tbl_hbm.at[pl.ds(t * TILE, TILE), :], out_sem)

    def wait_writeback(buf):
        pltpu.make_async_copy(out_v.at[buf, :, pl.ds(0, DIMP)],
                              tbl_hbm.at[pl.ds(0, TILE), :], out_sem).wait()

    def blk_index(i):
        return wid + i * NW

    nmine = ROUNDS + jnp.where(wid < EXTRA, 1, 0)
    stage(blk_index(0), 0)

    def step(i, _):
        buf = lax.rem(i, 2)
        nxt = lax.rem(i + 1, 2)

        @pl.when(i + 1 < nmine)
        def _():
            @pl.when(i + 1 >= 2)
            def _():
                wait_writeback(nxt)
            stage(blk_index(i + 1), nxt)

        wait_stage(buf)
        transpose(buf)
        writeback(blk_index(i), buf)
        return _

    lax.fori_loop(0, nmine, step, None, unroll=False)
    wait_writeback(0)
    wait_writeback(1)

    @pl.when(wid == 4)
    def _():
        pltpu.sync_copy(wtail_hbm, blk_v.at[0])
        transpose(0)
        pltpu.sync_copy(out_v.at[0, :, pl.ds(0, DIMP)],
                        tbl_hbm.at[pl.ds(NFULL * TILE, TILE), :])


SPAN = 1024
NPAIR = B_ROWS // SPAN
SPS = 25
IB = 128
CHUNK = 256
KC = CHUNK // IB
CPS = SPAN // CHUNK
NCHUNK = SPS * CPS
NBUF = 2
NGROUP = NCHUNK // NBUF
IROWS = SPAN // IB


@functools.partial(
    pl.kernel,
    **_MESH,
    out_type=jax.ShapeDtypeStruct((B_COLS, DIM, B_ROWS), jnp.float32),
    scratch_types=[
        pltpu.VMEM((NBUF, IROWS, IB), jnp.int32),
        pltpu.VMEM((NBUF, CHUNK, DIMP), jnp.float32),
        pltpu.VMEM((NBUF, DIM, CHUNK + 1), jnp.float32),
        pltpu.SemaphoreType.DMA,
        pltpu.SemaphoreType.DMA,
    ],
    compiler_params=_PARAMS,
)
def _lookup_kernel(idx_hbm, tbl_hbm, out_hbm, idx_v, rows_v, t_v,
                   gat_sem, out_sem):
    wid = lax.axis_index("s") * NC + lax.axis_index("c")
    pair = wid // 2
    half = wid % 2
    d0 = pair * SPAN
    s0 = half * SPS
    iotas = [lax.iota(jnp.int32, 16) + 16 * j for j in range(16)]

    def stage_idx(c, buf):
        s = s0 + c // CPS
        pltpu.sync_copy(idx_hbm.at[pl.ds(s * (B_ROWS // IB) + pair * IROWS,
                                         IROWS), :],
                        idx_v.at[buf])

    def start_gather(c, buf):
        stage_idx(c, buf)
        k = c % CPS
        for j in range(KC):
            pltpu.async_copy(
                tbl_hbm.at[idx_v.at[buf, k * KC + j]],
                rows_v.at[buf, pl.ds(j * IB, IB)],
                gat_sem,
            )

    def wait_gather(c, buf):
        k = c % CPS
        for j in range(KC):
            pltpu.make_async_copy(
                tbl_hbm.at[idx_v.at[buf, k * KC + j]],
                rows_v.at[buf, pl.ds(j * IB, IB)],
                gat_sem,
            ).wait()

    def transpose(buf):
        def row(l, _):
            lsplat = jnp.full((16,), l, jnp.int32)
            for j in range(4):
                v = rows_v[buf, l, pl.ds(16 * j, 16)]
                plsc.store_scatter(t_v.at[buf], [iotas[j], lsplat], v)
            return _

        lax.fori_loop(0, CHUNK, row, None, unroll=4)

    def writeback(c, buf):
        s = s0 + c // CPS
        off = d0 + (c % CPS) * CHUNK
        pltpu.async_copy(t_v.at[buf, :, pl.ds(0, CHUNK)],
                         out_hbm.at[s, :, pl.ds(off, CHUNK)], out_sem)

    def wait_writeback(buf):
        pltpu.make_async_copy(t_v.at[buf, :, pl.ds(0, CHUNK)],
                              out_hbm.at[0, :, pl.ds(0, CHUNK)],
                              out_sem).wait()

    start_gather(0, 0)

    def group(g, _):
        c0 = g * NBUF
        for b in range(NBUF):
            c = c0 + b
            nb = (b + 1) % NBUF

            @pl.when(c + 1 < NCHUNK)
            def _():
                start_gather(c + 1, nb)

            wait_gather(c, b)

            @pl.when(c >= NBUF)
            def _():
                wait_writeback(b)

            transpose(b)
            writeback(c, b)
        return _

    lax.fori_loop(0, NGROUP, group, None, unroll=False)

    for b in range(NBUF):
        wait_writeback(b)


def kernel(input_, weight):
    idx = input_.T.astype(jnp.int32).reshape(B_ROWS * B_COLS // IB, IB)
    wT = weight.T
    wtail = jnp.pad(wT[:, NFULL * TILE:], ((0, 0), (0, TILE - TAIL)))
    tbl = _relayout_kernel(wT, wtail)
    out3 = _lookup_kernel(idx, tbl)
    return jnp.transpose(out3, (2, 0, 1))


# speedup vs baseline: 1.3648x; 1.3648x over previous
import functools

import jax
import jax.numpy as jnp
from jax import lax
from jax.experimental import pallas as pl
from jax.experimental.pallas import tpu as pltpu
from jax.experimental.pallas import tpu_sc as plsc

DIM = 64
DIMP = 128
B_ROWS = 16384
B_COLS = 50
VOCAB = 1000000
TILE = 128
NFULL = VOCAB // TILE
TAIL = VOCAB - NFULL * TILE
VOCABP = (NFULL + 1) * TILE

_info = plsc.get_sparse_core_info()
NC = _info.num_cores
NS = _info.num_subcores
NW = NC * NS

_MESH = dict(mesh=plsc.VectorSubcoreMesh(core_axis_name="c", subcore_axis_name="s"))
_PARAMS = pltpu.CompilerParams(use_tc_tiling_on_sc=True,
                               needs_layout_passes=False)

ROUNDS = NFULL // NW
EXTRA = NFULL - ROUNDS * NW


@functools.partial(
    pl.kernel,
    **_MESH,
    out_type=jax.ShapeDtypeStruct((VOCABP, DIMP), jnp.float32),
    scratch_types=[
        pltpu.VMEM((2, DIM, TILE), jnp.float32),
        pltpu.VMEM((2, TILE, DIMP + 1), jnp.float32),
        pltpu.SemaphoreType.DMA,
        pltpu.SemaphoreType.DMA,
    ],
    compiler_params=_PARAMS,
)
def _relayout_kernel(wt_hbm, wtail_hbm, tbl_hbm, blk_v, out_v, in_sem, out_sem):
    wid = lax.axis_index("s") * NC + lax.axis_index("c")
    iotas = [lax.iota(jnp.int32, 16) + 16 * j for j in range(8)]

    def stage(t, buf):
        pltpu.async_copy(wt_hbm.at[:, pl.ds(t * TILE, TILE)], blk_v.at[buf],
                         in_sem)

    def wait_stage(buf):
        pltpu.make_async_copy(wt_hbm.at[:, pl.ds(0, TILE)], blk_v.at[buf],
                              in_sem).wait()

    def transpose(buf):
        @plsc.parallel_loop(0, DIM, unroll=8)
        def _comp(c):
            csplat = jnp.full((16,), c, jnp.int32)
            for j in range(8):
                v = blk_v[buf, c, pl.ds(16 * j, 16)]
                plsc.store_scatter(out_v.at[buf], [iotas[j], csplat], v)

    def writeback(t, buf):
        pltpu.async_copy(out_v.at[buf, :, pl.ds(0, DIMP)],
                         tbl_hbm.at[pl.ds(t * TILE, TILE), :], out_sem)

    def wait_writeback(buf):
        pltpu.make_async_copy(out_v.at[buf, :, pl.ds(0, DIMP)],
                              tbl_hbm.at[pl.ds(0, TILE), :], out_sem).wait()

    def blk_index(i):
        return wid + i * NW

    nmine = ROUNDS + jnp.where(wid < EXTRA, 1, 0)
    stage(blk_index(0), 0)

    def step(i, _):
        buf = lax.rem(i, 2)
        nxt = lax.rem(i + 1, 2)

        @pl.when(i + 1 < nmine)
        def _():
            @pl.when(i + 1 >= 2)
            def _():
                wait_writeback(nxt)
            stage(blk_index(i + 1), nxt)

        wait_stage(buf)
        transpose(buf)
        writeback(blk_index(i), buf)
        return _

    lax.fori_loop(0, nmine, step, None, unroll=False)
    wait_writeback(0)
    wait_writeback(1)

    @pl.when(wid == 4)
    def _():
        pltpu.sync_copy(wtail_hbm, blk_v.at[0])
        transpose(0)
        pltpu.sync_copy(out_v.at[0, :, pl.ds(0, DIMP)],
                        tbl_hbm.at[pl.ds(NFULL * TILE, TILE), :])


SPAN = 1024
NPAIR = B_ROWS // SPAN
SPS = 25
IB = 128
CHUNK = 256
KC = CHUNK // IB
CPS = SPAN // CHUNK
NCHUNK = SPS * CPS
NBUF = 2
NGROUP = NCHUNK // NBUF
IROWS = SPAN // IB


@functools.partial(
    pl.kernel,
    **_MESH,
    out_type=jax.ShapeDtypeStruct((B_COLS, DIM, B_ROWS), jnp.float32),
    scratch_types=[
        pltpu.VMEM((NBUF, IROWS, IB), jnp.int32),
        pltpu.VMEM((NBUF, CHUNK, DIMP), jnp.float32),
        pltpu.VMEM((NBUF, DIM, CHUNK + 1), jnp.float32),
        pltpu.SemaphoreType.DMA,
        pltpu.SemaphoreType.DMA,
    ],
    compiler_params=_PARAMS,
)
def _lookup_kernel(idx_hbm, tbl_hbm, out_hbm, idx_v, rows_v, t_v,
                   gat_sem, out_sem):
    wid = lax.axis_index("s") * NC + lax.axis_index("c")
    pair = wid // 2
    half = wid % 2
    d0 = pair * SPAN
    s0 = half * SPS
    iotas = [lax.iota(jnp.int32, 16) + 16 * j for j in range(16)]

    def stage_idx(c, buf):
        s = s0 + c // CPS
        pltpu.sync_copy(idx_hbm.at[pl.ds(s * (B_ROWS // IB) + pair * IROWS,
                                         IROWS), :],
                        idx_v.at[buf])

    def start_gather(c, buf):
        stage_idx(c, buf)
        k = c % CPS
        for j in range(KC):
            pltpu.async_copy(
                tbl_hbm.at[idx_v.at[buf, k * KC + j]],
                rows_v.at[buf, pl.ds(j * IB, IB)],
                gat_sem,
            )

    def wait_gather(c, buf):
        k = c % CPS
        for j in range(KC):
            pltpu.make_async_copy(
                tbl_hbm.at[idx_v.at[buf, k * KC + j]],
                rows_v.at[buf, pl.ds(j * IB, IB)],
                gat_sem,
            ).wait()

    def transpose(buf):
        @plsc.parallel_loop(0, CHUNK, unroll=8)
        def _row(l):
            lsplat = jnp.full((16,), l, jnp.int32)
            for j in range(4):
                v = rows_v[buf, l, pl.ds(16 * j, 16)]
                plsc.store_scatter(t_v.at[buf], [iotas[j], lsplat], v)

    def writeback(c, buf):
        s = s0 + c // CPS
        off = d0 + (c % CPS) * CHUNK
        pltpu.async_copy(t_v.at[buf, :, pl.ds(0, CHUNK)],
                         out_hbm.at[s, :, pl.ds(off, CHUNK)], out_sem)

    def wait_writeback(buf):
        pltpu.make_async_copy(t_v.at[buf, :, pl.ds(0, CHUNK)],
                              out_hbm.at[0, :, pl.ds(0, CHUNK)],
                              out_sem).wait()

    start_gather(0, 0)

    def group(g, _):
        c0 = g * NBUF
        for b in range(NBUF):
            c = c0 + b
            nb = (b + 1) % NBUF

            @pl.when(c + 1 < NCHUNK)
            def _():
                start_gather(c + 1, nb)

            wait_gather(c, b)

            @pl.when(c >= NBUF)
            def _():
                wait_writeback(b)

            transpose(b)
            writeback(c, b)
        return _

    lax.fori_loop(0, NGROUP, group, None, unroll=False)

    for b in range(NBUF):
        wait_writeback(b)


def kernel(input_, weight):
    idx = input_.T.astype(jnp.int32).reshape(B_ROWS * B_COLS // IB, IB)
    wT = weight.T
    wtail = jnp.pad(wT[:, NFULL * TILE:], ((0, 0), (0, TILE - TAIL)))
    tbl = _relayout_kernel(wT, wtail)
    out3 = _lookup_kernel(idx, tbl)
    return jnp.transpose(out3, (2, 0, 1))


# speedup vs baseline: 1.9206x; 1.4072x over previous
import functools

import jax
import jax.numpy as jnp
from jax import lax
from jax.experimental import pallas as pl
from jax.experimental.pallas import tpu as pltpu
from jax.experimental.pallas import tpu_sc as plsc

DIM = 64
B_ROWS = 16384
B_COLS = 50

_info = plsc.get_sparse_core_info()
NC = _info.num_cores
NS = _info.num_subcores
NW = NC * NS
D0_PER_W = B_ROWS // NW

IB = 128
K = D0_PER_W // IB
NCHUNK = B_COLS
NBUF = 2
NGROUP = NCHUNK // NBUF


@functools.partial(
    pl.kernel,
    mesh=plsc.VectorSubcoreMesh(core_axis_name="c", subcore_axis_name="s"),
    out_type=jax.ShapeDtypeStruct((B_ROWS, B_COLS, DIM), jnp.float32),
    scratch_types=[
        pltpu.VMEM((NBUF, D0_PER_W), jnp.int32),
        pltpu.VMEM((NBUF, D0_PER_W, DIM), jnp.float32),
        pltpu.SemaphoreType.DMA,
        pltpu.SemaphoreType.DMA,
    ],
    compiler_params=pltpu.CompilerParams(use_tc_tiling_on_sc=False),
)
def _gather_kernel(idxT_hbm, table_hbm, out_hbm, idx_v, rows_v, gat_sem, out_sem):
    wid = lax.axis_index("s") * NC + lax.axis_index("c")
    d0 = wid * D0_PER_W

    def start_gather(s, buf):
        pltpu.sync_copy(idxT_hbm.at[s, pl.ds(d0, D0_PER_W)], idx_v.at[buf])
        for j in range(K):
            pltpu.async_copy(
                table_hbm.at[idx_v.at[buf, pl.ds(j * IB, IB)]],
                rows_v.at[buf, pl.ds(j * IB, IB)],
                gat_sem,
            )

    def wait_gather(s, buf):
        for j in range(K):
            pltpu.make_async_copy(
                table_hbm.at[idx_v.at[buf, pl.ds(j * IB, IB)]],
                rows_v.at[buf, pl.ds(j * IB, IB)],
                gat_sem,
            ).wait()

    def wait_writeback(buf):
        pltpu.make_async_copy(
            rows_v.at[buf], out_hbm.at[pl.ds(d0, D0_PER_W), 0, :], out_sem
        ).wait()

    start_gather(0, 0)

    def group(g, _):
        s0 = g * NBUF
        for b in range(NBUF):
            s = s0 + b
            nb = (b + 1) % NBUF

            @pl.when(s + 1 < NCHUNK)
            def _():
                @pl.when(s + 1 >= NBUF)
                def _():
                    wait_writeback(nb)
                start_gather(s + 1, nb)

            wait_gather(s, b)
            pltpu.async_copy(
                rows_v.at[b], out_hbm.at[pl.ds(d0, D0_PER_W), s, :], out_sem
            )
        return _

    lax.fori_loop(0, NGROUP, group, None, unroll=False)

    for b in range(NBUF):
        wait_writeback(b)


def kernel(input_, weight):
    idxT = input_.T.astype(jnp.int32)
    return _gather_kernel(idxT, weight)
